# baseline (speedup 1.0000x reference)
import functools

import jax
import jax.numpy as jnp
from jax import lax
from jax.experimental import pallas as pl
from jax.experimental.pallas import tpu as pltpu
from jax.experimental.pallas import tpu_sc as plsc

BATCH = 16384
EMB_DIM = 32
NC = 2
NS = 16
NW = NC * NS
BPW = BATCH // NW


def _distmult_body(h_hbm, r_hbm, t_hbm, ent_hbm, rel_hbm, out_hbm,
                   hidx_v, ridx_v, tidx_v, eh_v, er_v, et_v, out_v, sem):
    wid = lax.axis_index("s") * NC + lax.axis_index("c")
    base = wid * BPW

    pltpu.sync_copy(h_hbm.at[pl.ds(base, BPW)], hidx_v)
    pltpu.sync_copy(r_hbm.at[pl.ds(base, BPW)], ridx_v)
    pltpu.sync_copy(t_hbm.at[pl.ds(base, BPW)], tidx_v)

    ch = pltpu.async_copy(ent_hbm.at[hidx_v], eh_v, sem)
    cr = pltpu.async_copy(rel_hbm.at[ridx_v], er_v, sem)
    ct = pltpu.async_copy(ent_hbm.at[tidx_v], et_v, sem)
    ch.wait()
    cr.wait()
    ct.wait()

    lanes = lax.iota(jnp.int32, 16)

    def group(g, _):
        rowbase = g * 16
        acc = jnp.zeros((16,), jnp.float32)
        for i in range(16):
            r = rowbase + i
            h0 = eh_v[r, pl.ds(0, 16)]
            h1 = eh_v[r, pl.ds(16, 16)]
            r0 = er_v[r, pl.ds(0, 16)]
            r1 = er_v[r, pl.ds(16, 16)]
            t0 = et_v[r, pl.ds(0, 16)]
            t1 = et_v[r, pl.ds(16, 16)]
            half = h0 * r0 * t0 + h1 * r1 * t1
            s = jnp.sum(half)
            acc = jnp.where(lanes == i, s, acc)
        out_v[pl.ds(rowbase, 16)] = acc
        return 0

    lax.fori_loop(0, BPW // 16, group, 0)

    pltpu.sync_copy(out_v, out_hbm.at[pl.ds(base, BPW)])


@jax.jit
def _distmult(hs, rs, ts, ent_embs, rel_embs):
    mesh = plsc.VectorSubcoreMesh(core_axis_name="c", subcore_axis_name="s")
    kern = functools.partial(
        pl.kernel,
        mesh=mesh,
        compiler_params=pltpu.CompilerParams(
            needs_layout_passes=False, use_tc_tiling_on_sc=False),
        out_type=jax.ShapeDtypeStruct((BATCH,), jnp.float32),
        scratch_types=[
            pltpu.VMEM((BPW,), jnp.int32),
            pltpu.VMEM((BPW,), jnp.int32),
            pltpu.VMEM((BPW,), jnp.int32),
            pltpu.VMEM((BPW, EMB_DIM), jnp.float32),
            pltpu.VMEM((BPW, EMB_DIM), jnp.float32),
            pltpu.VMEM((BPW, EMB_DIM), jnp.float32),
            pltpu.VMEM((BPW,), jnp.float32),
            pltpu.SemaphoreType.DMA,
        ],
    )(_distmult_body)
    return kern(hs, rs, ts, ent_embs, rel_embs)


def kernel(batch, ent_embs, rel_embs):
    hs = batch[:, 0]
    rs = batch[:, 1]
    ts = batch[:, 2]
    return _distmult(hs, rs, ts, ent_embs, rel_embs)
